# baseline (device time: 171018 ns/iter reference)
import jax
import jax.numpy as jnp
from jax import lax
from jax.experimental import pallas as pl
from jax.experimental.pallas import tpu as pltpu

N_DEV = 16
B, SQ, DM = 2, 256, 768
HQ_PER = 8
DH = 64
SCALE = 0.125


def kernel(x, Wq, Wo, Wk, Wv):
    i = lax.axis_index("i")
    wk = lax.dynamic_slice_in_dim(Wk, i * 2 * DH, 2 * DH, axis=1)
    wv = lax.dynamic_slice_in_dim(Wv, i * 2 * DH, 2 * DH, axis=1)

    def body(x_ref, wq_ref, wo_ref, wk_ref, wv_ref, out_ref,
             comm_ref, send_sems, recv_sems):
        my = lax.axis_index("i")
        left = (my - 1) % N_DEV
        right = (my + 1) % N_DEV

        barrier = pltpu.get_barrier_semaphore()
        for nbr in (left, right):
            pl.semaphore_signal(barrier, inc=1, device_id=(nbr,),
                                device_id_type=pl.DeviceIdType.MESH)
        pl.semaphore_wait(barrier, 2)

        wq = wq_ref[...].astype(jnp.bfloat16)
        wk = wk_ref[...].astype(jnp.bfloat16)
        wv = wv_ref[...].astype(jnp.bfloat16)
        wo = wo_ref[...].astype(jnp.bfloat16)

        for b in range(B):
            xb = x_ref[b, :, :].astype(jnp.bfloat16)
            q = jnp.dot(xb, wq, preferred_element_type=jnp.float32)
            q = (q * SCALE).astype(jnp.bfloat16)
            k = jnp.dot(xb, wk, preferred_element_type=jnp.float32
                        ).astype(jnp.bfloat16)
            v = jnp.dot(xb, wv, preferred_element_type=jnp.float32
                        ).astype(jnp.bfloat16)
            heads = []
            for h in range(HQ_PER):
                kv = h // 4
                qh = q[:, h * DH:(h + 1) * DH]
                kh = k[:, kv * DH:(kv + 1) * DH]
                vh = v[:, kv * DH:(kv + 1) * DH]
                s = lax.dot_general(qh, kh, (((1,), (1,)), ((), ())),
                                    preferred_element_type=jnp.float32)
                m = jnp.max(s, axis=1, keepdims=True)
                p = jnp.exp(s - m)
                l = jnp.sum(p, axis=1, keepdims=True)
                o = jnp.dot(p.astype(jnp.bfloat16), vh,
                            preferred_element_type=jnp.float32) / l
                heads.append(o.astype(jnp.bfloat16))
            attn = jnp.concatenate(heads, axis=1)
            partial = jnp.dot(attn, wo, preferred_element_type=jnp.float32)
            out_ref[b, :, :] = partial
            comm_ref[0, b, :, :] = partial.astype(jnp.bfloat16)

        for hop in range(N_DEV - 1):
            s_slot = hop % 2
            r_slot = (hop + 1) % 2
            rdma = pltpu.make_async_remote_copy(
                src_ref=comm_ref.at[s_slot],
                dst_ref=comm_ref.at[r_slot],
                send_sem=send_sems.at[s_slot],
                recv_sem=recv_sems.at[r_slot],
                device_id=(right,),
                device_id_type=pl.DeviceIdType.MESH,
            )
            rdma.start()
            rdma.wait()
            out_ref[...] += comm_ref[r_slot].astype(jnp.float32)

    return pl.pallas_call(
        body,
        out_shape=jax.ShapeDtypeStruct((B, SQ, DM), jnp.float32),
        in_specs=[pl.BlockSpec(memory_space=pltpu.VMEM)] * 5,
        out_specs=pl.BlockSpec(memory_space=pltpu.VMEM),
        scratch_shapes=[
            pltpu.VMEM((2, B, SQ, DM), jnp.bfloat16),
            pltpu.SemaphoreType.DMA((2,)),
            pltpu.SemaphoreType.DMA((2,)),
        ],
        compiler_params=pltpu.CompilerParams(collective_id=0),
    )(x, Wq, Wo, wk, wv)


# device time: 13233 ns/iter; 12.9236x vs baseline; 12.9236x over previous
import jax
import jax.numpy as jnp
from jax import lax
from jax.experimental import pallas as pl
from jax.experimental.pallas import tpu as pltpu

N_DEV = 16
B, SQ, DM = 2, 256, 768
ROWS = B * SQ
CH = ROWS // N_DEV
HQ_PER = 8
DH = 64
SCALE = 0.125


def kernel(x, Wq, Wo, Wk, Wv):
    i = lax.axis_index("i")
    wk = lax.dynamic_slice_in_dim(Wk, i * 2 * DH, 2 * DH, axis=1)
    wv = lax.dynamic_slice_in_dim(Wv, i * 2 * DH, 2 * DH, axis=1)
    xf = x.reshape(ROWS, DM)

    def body(x_ref, wq_ref, wo_ref, wk_ref, wv_ref, out_ref,
             sbuf, rbuf, red, abuf, rs_send, rs_recv, ag_send, ag_recv):
        my = lax.axis_index("i")

        barrier = pltpu.get_barrier_semaphore()
        for k in range(1, N_DEV):
            pl.semaphore_signal(barrier, inc=1, device_id=((my + k) % N_DEV,),
                                device_id_type=pl.DeviceIdType.MESH)
        pl.semaphore_wait(barrier, N_DEV - 1)

        wq = wq_ref[...].astype(jnp.bfloat16)
        wkv_k = wk_ref[...].astype(jnp.bfloat16)
        wkv_v = wv_ref[...].astype(jnp.bfloat16)
        wo = wo_ref[...].astype(jnp.bfloat16)

        xall = x_ref[...].astype(jnp.bfloat16)
        q = jnp.dot(xall, wq, preferred_element_type=jnp.float32)
        q = (q * SCALE).astype(jnp.bfloat16)
        kk = jnp.dot(xall, wkv_k, preferred_element_type=jnp.float32
                     ).astype(jnp.bfloat16)
        vv = jnp.dot(xall, wkv_v, preferred_element_type=jnp.float32
                     ).astype(jnp.bfloat16)

        batches = []
        for b in range(B):
            r0, r1 = b * SQ, (b + 1) * SQ
            heads = []
            for h in range(HQ_PER):
                kv = h // 4
                qh = q[r0:r1, h * DH:(h + 1) * DH]
                kh = kk[r0:r1, kv * DH:(kv + 1) * DH]
                vh = vv[r0:r1, kv * DH:(kv + 1) * DH]
                s = lax.dot_general(qh, kh, (((1,), (1,)), ((), ())),
                                    preferred_element_type=jnp.float32)
                m = jnp.max(s, axis=1, keepdims=True)
                p = jnp.exp(s - m)
                l = jnp.sum(p, axis=1, keepdims=True)
                o = jnp.dot(p.astype(jnp.bfloat16), vh,
                            preferred_element_type=jnp.float32) / l
                heads.append(o.astype(jnp.bfloat16))
            batches.append(jnp.concatenate(heads, axis=1))
        attn = jnp.concatenate(batches, axis=0)
        partial = jnp.dot(attn, wo, preferred_element_type=jnp.float32)
        out_ref[...] = partial

        rs_descs = []
        for k in range(1, N_DEV):
            dst = (my + k) % N_DEV
            sbuf[k - 1, :, :] = out_ref[pl.ds(dst * CH, CH), :
                                        ].astype(jnp.bfloat16)
            d = pltpu.make_async_remote_copy(
                src_ref=sbuf.at[k - 1],
                dst_ref=rbuf.at[k - 1],
                send_sem=rs_send.at[k - 1],
                recv_sem=rs_recv.at[k - 1],
                device_id=(dst,),
                device_id_type=pl.DeviceIdType.MESH,
            )
            d.start()
            rs_descs.append(d)

        acc = out_ref[pl.ds(my * CH, CH), :]
        for k in range(1, N_DEV):
            rs_descs[k - 1].wait_recv()
            acc = acc + rbuf[k - 1, :, :].astype(jnp.float32)
        red[...] = acc.astype(jnp.bfloat16)

        ag_descs = []
        for k in range(1, N_DEV):
            dst = (my + k) % N_DEV
            d = pltpu.make_async_remote_copy(
                src_ref=red,
                dst_ref=abuf.at[k - 1],
                send_sem=ag_send.at[k - 1],
                recv_sem=ag_recv.at[k - 1],
                device_id=(dst,),
                device_id_type=pl.DeviceIdType.MESH,
            )
            d.start()
            ag_descs.append(d)
        out_ref[pl.ds(my * CH, CH), :] = acc
        for k in range(1, N_DEV):
            ag_descs[k - 1].wait_recv()
            src_dev = (my - k) % N_DEV
            out_ref[pl.ds(src_dev * CH, CH), :] = abuf[k - 1, :, :
                                                       ].astype(jnp.float32)
        for k in range(1, N_DEV):
            rs_descs[k - 1].wait_send()
            ag_descs[k - 1].wait_send()

    out = pl.pallas_call(
        body,
        out_shape=jax.ShapeDtypeStruct((ROWS, DM), jnp.float32),
        in_specs=[pl.BlockSpec(memory_space=pltpu.VMEM)] * 5,
        out_specs=pl.BlockSpec(memory_space=pltpu.VMEM),
        scratch_shapes=[
            pltpu.VMEM((N_DEV - 1, CH, DM), jnp.bfloat16),
            pltpu.VMEM((N_DEV - 1, CH, DM), jnp.bfloat16),
            pltpu.VMEM((CH, DM), jnp.bfloat16),
            pltpu.VMEM((N_DEV - 1, CH, DM), jnp.bfloat16),
            pltpu.SemaphoreType.DMA((N_DEV - 1,)),
            pltpu.SemaphoreType.DMA((N_DEV - 1,)),
            pltpu.SemaphoreType.DMA((N_DEV - 1,)),
            pltpu.SemaphoreType.DMA((N_DEV - 1,)),
        ],
        compiler_params=pltpu.CompilerParams(collective_id=0),
    )(xf, Wq, Wo, wk, wv)
    return out.reshape(B, SQ, DM)
